# Initial kernel scaffold; baseline (speedup 1.0000x reference)
import jax
import jax.numpy as jnp
from jax.experimental import pallas as pl


def kernel(loc, conf, iou):
    raise NotImplementedError("write your pallas kernel here")


# speedup vs baseline: 38.4130x; 38.4130x over previous
import math

import numpy as np
import jax
import jax.numpy as jnp
from jax.experimental import pallas as pl
from jax.experimental.pallas import tpu as pltpu

_H = 512
_W = 512
_MIN_SIZES = [[10, 16, 24], [32, 48], [64, 96], [128, 192, 256]]
_STEPS = [8, 16, 32, 64]
_VAR0 = 0.1
_VAR1 = 0.2
_TOP_K = 5000
_CONF_THR = 0.3
_NMS_THR = 0.3
_KEEP_TOP_K = 750

_NP = sum(math.ceil(_H / s) * math.ceil(_W / s) * len(m)
          for s, m in zip(_STEPS, _MIN_SIZES))
_NP_PAD = ((_NP + 127) // 128) * 128
_T = 128
_N_PAD = ((_TOP_K + _T - 1) // _T) * _T
_NB = _N_PAD // _T


def _build_priors_np():
    anchors = []
    for k, step in enumerate(_STEPS):
        fh = math.ceil(_H / step)
        fw = math.ceil(_W / step)
        for i in range(fh):
            for j in range(fw):
                for ms in _MIN_SIZES[k]:
                    anchors.append([(j + 0.5) * step / _W, (i + 0.5) * step / _H,
                                    ms / _W, ms / _H])
    return np.array(anchors, dtype=np.float32)


_PRIORS_T = np.zeros((4, _NP_PAD), dtype=np.float32)
_PRIORS_T[:, :_NP] = _build_priors_np().T


def _decode_kernel(loc_ref, pri_ref, conf_ref, iou_ref, box_ref, sc_ref):
    loc = loc_ref[...]
    pxy = pri_ref[0:2, :]
    pwh = pri_ref[2:4, :]
    cxy = pxy + loc[0:2, :] * _VAR0 * pwh
    wh = pwh * jnp.exp(loc[2:4, :] * _VAR1)
    x1y1 = cxy - wh * 0.5
    x2y2 = x1y1 + wh
    lmks = [pxy + loc[s:s + 2, :] * _VAR0 * pwh for s in (4, 6, 8, 10, 12)]
    box_ref[...] = jnp.concatenate([x1y1, x2y2] + lmks, axis=0) * float(_W)
    cls = conf_ref[...]
    iouv = jnp.clip(iou_ref[...], 0.0, 1.0)
    s = jnp.sqrt(jnp.maximum(cls * iouv, 1e-12))
    sc_ref[...] = jnp.where(s > _CONF_THR, s, -1.0)


def _nms_kernel(rows_ref, cols_ref, valid_ref, keep_ref, mb_ref, kb_ref):
    x1 = cols_ref[0:1, :]
    y1 = cols_ref[1:2, :]
    x2 = cols_ref[2:3, :]
    y2 = cols_ref[3:4, :]
    areas = (x2 - x1) * (y2 - y1)
    iota_n = jax.lax.broadcasted_iota(jnp.int32, (1, _N_PAD), 1)
    iota_t = jax.lax.broadcasted_iota(jnp.int32, (1, _T), 1)
    keep_ref[...] = valid_ref[...]

    def block_body(i, carry):
        bb = rows_ref[pl.dslice(i * _T, _T), :]
        x1b = bb[:, 0:1]
        y1b = bb[:, 1:2]
        x2b = bb[:, 2:3]
        y2b = bb[:, 3:4]
        ab = (x2b - x1b) * (y2b - y1b)
        inter = (jnp.maximum(jnp.minimum(x2b, x2) - jnp.maximum(x1b, x1), 0.0)
                 * jnp.maximum(jnp.minimum(y2b, y2) - jnp.maximum(y1b, y1), 0.0))
        iou = inter / (ab + areas - inter + 1e-12)
        m = (iou > _NMS_THR).astype(jnp.float32)

        x1r = cols_ref[0:1, pl.dslice(i * _T, _T)]
        y1r = cols_ref[1:2, pl.dslice(i * _T, _T)]
        x2r = cols_ref[2:3, pl.dslice(i * _T, _T)]
        y2r = cols_ref[3:4, pl.dslice(i * _T, _T)]
        ar = (x2r - x1r) * (y2r - y1r)
        interb = (jnp.maximum(jnp.minimum(x2b, x2r) - jnp.maximum(x1b, x1r), 0.0)
                  * jnp.maximum(jnp.minimum(y2b, y2r) - jnp.maximum(y1b, y1r), 0.0))
        ioub = interb / (ab + ar - interb + 1e-12)
        mb_ref[...] = (ioub > _NMS_THR).astype(jnp.float32)
        kb_ref[...] = keep_ref[0:1, pl.dslice(i * _T, _T)]

        def intra(t, c):
            row = mb_ref[pl.dslice(t, 1), :]
            kb = kb_ref[...]
            piv = jnp.sum(kb * (iota_t == t).astype(jnp.float32))
            gt = (iota_t > t).astype(jnp.float32)
            kb_ref[...] = kb * (1.0 - row * piv * gt)
            return c

        jax.lax.fori_loop(0, _T, intra, 0)
        kb = kb_ref[...]
        keep_ref[0:1, pl.dslice(i * _T, _T)] = kb
        cross = jnp.dot(kb, m, preferred_element_type=jnp.float32)
        sup = (cross > 0.0) & (iota_n >= (i + 1) * _T)
        keep_ref[...] = keep_ref[...] * (1.0 - sup.astype(jnp.float32))
        return carry

    jax.lax.fori_loop(0, _NB, block_body, 0)


def kernel(loc, conf, iou):
    b = loc.shape[0]
    loc_t = jnp.transpose(loc, (0, 2, 1))
    loc_t = jnp.pad(loc_t, ((0, 0), (0, 0), (0, _NP_PAD - _NP)))
    conf1 = jnp.pad(conf[:, :, 1:2], ((0, 0), (0, _NP_PAD - _NP), (0, 0)))
    conf1 = jnp.transpose(conf1, (0, 2, 1))
    iou0 = jnp.pad(iou[:, :, 0:1], ((0, 0), (0, _NP_PAD - _NP), (0, 0)))
    iou0 = jnp.transpose(iou0, (0, 2, 1))
    priors = jnp.asarray(_PRIORS_T)

    boxes_t, scores = pl.pallas_call(
        _decode_kernel,
        grid=(b,),
        in_specs=[
            pl.BlockSpec((None, 14, _NP_PAD), lambda i: (i, 0, 0)),
            pl.BlockSpec((4, _NP_PAD), lambda i: (0, 0)),
            pl.BlockSpec((None, 1, _NP_PAD), lambda i: (i, 0, 0)),
            pl.BlockSpec((None, 1, _NP_PAD), lambda i: (i, 0, 0)),
        ],
        out_specs=[
            pl.BlockSpec((None, 14, _NP_PAD), lambda i: (i, 0, 0)),
            pl.BlockSpec((None, 1, _NP_PAD), lambda i: (i, 0, 0)),
        ],
        out_shape=[
            jax.ShapeDtypeStruct((b, 14, _NP_PAD), jnp.float32),
            jax.ShapeDtypeStruct((b, 1, _NP_PAD), jnp.float32),
        ],
    )(loc_t, priors, conf1, iou0)

    scores = scores[:, 0, :_NP]
    top_scores, order = jax.lax.top_k(scores, _TOP_K)
    boxes_full = jnp.transpose(boxes_t, (0, 2, 1))[:, :_NP, :]
    boxes_k = jnp.take_along_axis(boxes_full, order[:, :, None], axis=1)

    rows = jnp.pad(boxes_k[:, :, :4], ((0, 0), (0, _N_PAD - _TOP_K), (0, 0)))
    cols = jnp.transpose(rows, (0, 2, 1))
    valid = (top_scores > 0.0).astype(jnp.float32)
    valid = jnp.pad(valid, ((0, 0), (0, _N_PAD - _TOP_K)))[:, None, :]

    keep = pl.pallas_call(
        _nms_kernel,
        grid=(b,),
        in_specs=[
            pl.BlockSpec((None, _N_PAD, 4), lambda i: (i, 0, 0)),
            pl.BlockSpec((None, 4, _N_PAD), lambda i: (i, 0, 0)),
            pl.BlockSpec((None, 1, _N_PAD), lambda i: (i, 0, 0)),
        ],
        out_specs=pl.BlockSpec((None, 1, _N_PAD), lambda i: (i, 0, 0)),
        out_shape=jax.ShapeDtypeStruct((b, 1, _N_PAD), jnp.float32),
        scratch_shapes=[
            pltpu.VMEM((_T, _T), jnp.float32),
            pltpu.VMEM((1, _T), jnp.float32),
        ],
    )(rows, cols, valid)

    kept = keep[:, 0, :_TOP_K] > 0.5
    final_scores = jnp.where(kept, top_scores, -1.0)
    fs, fidx = jax.lax.top_k(final_scores, _KEEP_TOP_K)
    dets = jnp.take_along_axis(boxes_k, fidx[:, :, None], axis=1)
    sc = jnp.take_along_axis(top_scores, fidx, axis=1)
    out = jnp.concatenate([dets, sc[:, :, None]], axis=-1)
    return out * (fs > 0.0).astype(out.dtype)[:, :, None]


# speedup vs baseline: 210.0728x; 5.4688x over previous
import math

import numpy as np
import jax
import jax.numpy as jnp
from jax.experimental import pallas as pl
from jax.experimental.pallas import tpu as pltpu

_H = 512
_W = 512
_MIN_SIZES = [[10, 16, 24], [32, 48], [64, 96], [128, 192, 256]]
_STEPS = [8, 16, 32, 64]
_VAR0 = 0.1
_VAR1 = 0.2
_TOP_K = 5000
_CONF_THR = 0.3
_NMS_THR = 0.3
_KEEP_TOP_K = 750

_NP = sum(math.ceil(_H / s) * math.ceil(_W / s) * len(m)
          for s, m in zip(_STEPS, _MIN_SIZES))
_NP_PAD = ((_NP + 127) // 128) * 128
_T = 128
_N_PAD = ((_TOP_K + _T - 1) // _T) * _T
_NB = _N_PAD // _T


def _build_priors_np():
    anchors = []
    for k, step in enumerate(_STEPS):
        fh = math.ceil(_H / step)
        fw = math.ceil(_W / step)
        for i in range(fh):
            for j in range(fw):
                for ms in _MIN_SIZES[k]:
                    anchors.append([(j + 0.5) * step / _W, (i + 0.5) * step / _H,
                                    ms / _W, ms / _H])
    return np.array(anchors, dtype=np.float32)


_PRIORS_T = np.zeros((4, _NP_PAD), dtype=np.float32)
_PRIORS_T[:, :_NP] = _build_priors_np().T


def _decode_kernel(loc_ref, pri_ref, conf_ref, iou_ref, box_ref, sc_ref):
    loc = loc_ref[...]
    pxy = pri_ref[0:2, :]
    pwh = pri_ref[2:4, :]
    cxy = pxy + loc[0:2, :] * _VAR0 * pwh
    wh = pwh * jnp.exp(loc[2:4, :] * _VAR1)
    x1y1 = cxy - wh * 0.5
    x2y2 = x1y1 + wh
    lmks = [pxy + loc[s:s + 2, :] * _VAR0 * pwh for s in (4, 6, 8, 10, 12)]
    box_ref[...] = jnp.concatenate([x1y1, x2y2] + lmks, axis=0) * float(_W)
    cls = conf_ref[...]
    iouv = jnp.clip(iou_ref[...], 0.0, 1.0)
    s = jnp.sqrt(jnp.maximum(cls * iouv, 1e-12))
    sc_ref[...] = jnp.where(s > _CONF_THR, s, -1.0)


def _nms_kernel(rows_ref, cols_ref, valid_ref, keep_ref):
    x1 = cols_ref[0:1, :]
    y1 = cols_ref[1:2, :]
    x2 = cols_ref[2:3, :]
    y2 = cols_ref[3:4, :]
    areas = (x2 - x1) * (y2 - y1)
    iota_n = jax.lax.broadcasted_iota(jnp.int32, (1, _N_PAD), 1)
    rt = jax.lax.broadcasted_iota(jnp.int32, (_T, _T), 0)
    ct = jax.lax.broadcasted_iota(jnp.int32, (_T, _T), 1)
    ltmask = (rt < ct).astype(jnp.float32)
    keep_ref[...] = valid_ref[...]

    def block_body(i, carry):
        bb = rows_ref[pl.dslice(i * _T, _T), :]
        x1b = bb[:, 0:1]
        y1b = bb[:, 1:2]
        x2b = bb[:, 2:3]
        y2b = bb[:, 3:4]
        ab = (x2b - x1b) * (y2b - y1b)

        x1r = cols_ref[0:1, pl.dslice(i * _T, _T)]
        y1r = cols_ref[1:2, pl.dslice(i * _T, _T)]
        x2r = cols_ref[2:3, pl.dslice(i * _T, _T)]
        y2r = cols_ref[3:4, pl.dslice(i * _T, _T)]
        ar = (x2r - x1r) * (y2r - y1r)
        interb = (jnp.maximum(jnp.minimum(x2b, x2r) - jnp.maximum(x1b, x1r), 0.0)
                  * jnp.maximum(jnp.minimum(y2b, y2r) - jnp.maximum(y1b, y1r), 0.0))
        ioub = interb / (ab + ar - interb + 1e-12)
        mlt = (ioub > _NMS_THR).astype(jnp.float32) * ltmask
        kb0 = keep_ref[0:1, pl.dslice(i * _T, _T)]

        def fcond(carry):
            return carry[1]

        def fbody(carry):
            kb, _ = carry
            sup = jnp.dot(kb, mlt, preferred_element_type=jnp.float32)
            knew = kb0 * jnp.where(sup > 0.0, 0.0, 1.0)
            return knew, jnp.any(knew != kb)

        kb, _ = jax.lax.while_loop(fcond, fbody, (kb0, True))
        keep_ref[0:1, pl.dslice(i * _T, _T)] = kb

        inter = (jnp.maximum(jnp.minimum(x2b, x2) - jnp.maximum(x1b, x1), 0.0)
                 * jnp.maximum(jnp.minimum(y2b, y2) - jnp.maximum(y1b, y1), 0.0))
        iou = inter / (ab + areas - inter + 1e-12)
        m = (iou > _NMS_THR).astype(jnp.float32)
        cross = jnp.dot(kb, m, preferred_element_type=jnp.float32)
        sup = (cross > 0.0) & (iota_n >= (i + 1) * _T)
        keep_ref[...] = keep_ref[...] * (1.0 - sup.astype(jnp.float32))
        return carry

    jax.lax.fori_loop(0, _NB, block_body, 0)


def kernel(loc, conf, iou):
    b = loc.shape[0]
    loc_t = jnp.transpose(loc, (0, 2, 1))
    loc_t = jnp.pad(loc_t, ((0, 0), (0, 0), (0, _NP_PAD - _NP)))
    conf1 = jnp.pad(conf[:, :, 1:2], ((0, 0), (0, _NP_PAD - _NP), (0, 0)))
    conf1 = jnp.transpose(conf1, (0, 2, 1))
    iou0 = jnp.pad(iou[:, :, 0:1], ((0, 0), (0, _NP_PAD - _NP), (0, 0)))
    iou0 = jnp.transpose(iou0, (0, 2, 1))
    priors = jnp.asarray(_PRIORS_T)

    boxes_t, scores = pl.pallas_call(
        _decode_kernel,
        grid=(b,),
        in_specs=[
            pl.BlockSpec((None, 14, _NP_PAD), lambda i: (i, 0, 0)),
            pl.BlockSpec((4, _NP_PAD), lambda i: (0, 0)),
            pl.BlockSpec((None, 1, _NP_PAD), lambda i: (i, 0, 0)),
            pl.BlockSpec((None, 1, _NP_PAD), lambda i: (i, 0, 0)),
        ],
        out_specs=[
            pl.BlockSpec((None, 14, _NP_PAD), lambda i: (i, 0, 0)),
            pl.BlockSpec((None, 1, _NP_PAD), lambda i: (i, 0, 0)),
        ],
        out_shape=[
            jax.ShapeDtypeStruct((b, 14, _NP_PAD), jnp.float32),
            jax.ShapeDtypeStruct((b, 1, _NP_PAD), jnp.float32),
        ],
    )(loc_t, priors, conf1, iou0)

    scores = scores[:, 0, :_NP]
    top_scores, order = jax.lax.top_k(scores, _TOP_K)
    boxes_full = jnp.transpose(boxes_t, (0, 2, 1))[:, :_NP, :]
    boxes_k = jnp.take_along_axis(boxes_full, order[:, :, None], axis=1)

    rows = jnp.pad(boxes_k[:, :, :4], ((0, 0), (0, _N_PAD - _TOP_K), (0, 0)))
    cols = jnp.transpose(rows, (0, 2, 1))
    valid = (top_scores > 0.0).astype(jnp.float32)
    valid = jnp.pad(valid, ((0, 0), (0, _N_PAD - _TOP_K)))[:, None, :]

    keep = pl.pallas_call(
        _nms_kernel,
        grid=(b,),
        in_specs=[
            pl.BlockSpec((None, _N_PAD, 4), lambda i: (i, 0, 0)),
            pl.BlockSpec((None, 4, _N_PAD), lambda i: (i, 0, 0)),
            pl.BlockSpec((None, 1, _N_PAD), lambda i: (i, 0, 0)),
        ],
        out_specs=pl.BlockSpec((None, 1, _N_PAD), lambda i: (i, 0, 0)),
        out_shape=jax.ShapeDtypeStruct((b, 1, _N_PAD), jnp.float32),
    )(rows, cols, valid)

    kept = keep[:, 0, :_TOP_K] > 0.5
    final_scores = jnp.where(kept, top_scores, -1.0)
    fs, fidx = jax.lax.top_k(final_scores, _KEEP_TOP_K)
    dets = jnp.take_along_axis(boxes_k, fidx[:, :, None], axis=1)
    sc = jnp.take_along_axis(top_scores, fidx, axis=1)
    out = jnp.concatenate([dets, sc[:, :, None]], axis=-1)
    return out * (fs > 0.0).astype(out.dtype)[:, :, None]


# speedup vs baseline: 213.2044x; 1.0149x over previous
import math

import numpy as np
import jax
import jax.numpy as jnp
from jax.experimental import pallas as pl
from jax.experimental.pallas import tpu as pltpu

_H = 512
_W = 512
_MIN_SIZES = [[10, 16, 24], [32, 48], [64, 96], [128, 192, 256]]
_STEPS = [8, 16, 32, 64]
_VAR0 = 0.1
_VAR1 = 0.2
_TOP_K = 5000
_CONF_THR = 0.3
_NMS_THR = 0.3
_KEEP_TOP_K = 750

_NP = sum(math.ceil(_H / s) * math.ceil(_W / s) * len(m)
          for s, m in zip(_STEPS, _MIN_SIZES))
_NP_PAD = ((_NP + 127) // 128) * 128
_T = 128
_N_PAD = ((_TOP_K + _T - 1) // _T) * _T
_NB = _N_PAD // _T
_C = 512
_NC = _N_PAD // _C


def _build_priors_np():
    anchors = []
    for k, step in enumerate(_STEPS):
        fh = math.ceil(_H / step)
        fw = math.ceil(_W / step)
        for i in range(fh):
            for j in range(fw):
                for ms in _MIN_SIZES[k]:
                    anchors.append([(j + 0.5) * step / _W, (i + 0.5) * step / _H,
                                    ms / _W, ms / _H])
    return np.array(anchors, dtype=np.float32)


_PRIORS_T = np.zeros((4, _NP_PAD), dtype=np.float32)
_PRIORS_T[:, :_NP] = _build_priors_np().T


def _decode_kernel(loc_ref, pri_ref, conf_ref, iou_ref, box_ref, sc_ref):
    loc = loc_ref[...]
    pxy = pri_ref[0:2, :]
    pwh = pri_ref[2:4, :]
    cxy = pxy + loc[0:2, :] * _VAR0 * pwh
    wh = pwh * jnp.exp(loc[2:4, :] * _VAR1)
    x1y1 = cxy - wh * 0.5
    x2y2 = x1y1 + wh
    lmks = [pxy + loc[s:s + 2, :] * _VAR0 * pwh for s in (4, 6, 8, 10, 12)]
    box_ref[...] = jnp.concatenate([x1y1, x2y2] + lmks, axis=0) * float(_W)
    cls = conf_ref[...]
    iouv = jnp.clip(iou_ref[...], 0.0, 1.0)
    s = jnp.sqrt(jnp.maximum(cls * iouv, 1e-12))
    sc_ref[...] = jnp.where(s > _CONF_THR, s, -1.0)


def _nms_kernel(rows_ref, cols_ref, valid_ref, keep_ref):
    x1 = cols_ref[0:1, :]
    y1 = cols_ref[1:2, :]
    x2 = cols_ref[2:3, :]
    y2 = cols_ref[3:4, :]
    iota_c = jax.lax.broadcasted_iota(jnp.int32, (1, _C), 1)
    rt = jax.lax.broadcasted_iota(jnp.int32, (_T, _T), 0)
    ct = jax.lax.broadcasted_iota(jnp.int32, (_T, _T), 1)
    ltmask = (rt < ct).astype(jnp.float32)
    keep_ref[...] = valid_ref[...]

    def block_body(i, carry):
        bb = rows_ref[pl.dslice(i * _T, _T), :]
        x1b = bb[:, 0:1]
        y1b = bb[:, 1:2]
        x2b = bb[:, 2:3]
        y2b = bb[:, 3:4]
        ab = (x2b - x1b) * (y2b - y1b)

        x1r = cols_ref[0:1, pl.dslice(i * _T, _T)]
        y1r = cols_ref[1:2, pl.dslice(i * _T, _T)]
        x2r = cols_ref[2:3, pl.dslice(i * _T, _T)]
        y2r = cols_ref[3:4, pl.dslice(i * _T, _T)]
        ar = (x2r - x1r) * (y2r - y1r)
        interb = (jnp.maximum(jnp.minimum(x2b, x2r) - jnp.maximum(x1b, x1r), 0.0)
                  * jnp.maximum(jnp.minimum(y2b, y2r) - jnp.maximum(y1b, y1r), 0.0))
        ioub = interb / (ab + ar - interb + 1e-12)
        mlt = (ioub > _NMS_THR).astype(jnp.float32) * ltmask
        kb0 = keep_ref[0:1, pl.dslice(i * _T, _T)]

        def fcond(carry):
            return carry[1]

        def fbody(carry):
            kb, _ = carry
            sup = jnp.dot(kb, mlt, preferred_element_type=jnp.float32)
            knew = kb0 * jnp.where(sup > 0.0, 0.0, 1.0)
            return knew, jnp.any(knew != kb)

        kb, _ = jax.lax.while_loop(fcond, fbody, (kb0, True))
        keep_ref[0:1, pl.dslice(i * _T, _T)] = kb

        def col_body(j, c2):
            x1c = cols_ref[0:1, pl.dslice(j * _C, _C)]
            y1c = cols_ref[1:2, pl.dslice(j * _C, _C)]
            x2c = cols_ref[2:3, pl.dslice(j * _C, _C)]
            y2c = cols_ref[3:4, pl.dslice(j * _C, _C)]
            ac = (x2c - x1c) * (y2c - y1c)
            ic = (jnp.maximum(jnp.minimum(x2b, x2c) - jnp.maximum(x1b, x1c), 0.0)
                  * jnp.maximum(jnp.minimum(y2b, y2c) - jnp.maximum(y1b, y1c), 0.0))
            iouc = ic / (ab + ac - ic + 1e-12)
            mc = (iouc > _NMS_THR).astype(jnp.float32)
            cross = jnp.dot(kb, mc, preferred_element_type=jnp.float32)
            sup = (cross > 0.0) & (iota_c + j * _C >= (i + 1) * _T)
            keep_ref[0:1, pl.dslice(j * _C, _C)] = (
                keep_ref[0:1, pl.dslice(j * _C, _C)]
                * (1.0 - sup.astype(jnp.float32)))
            return c2

        jax.lax.fori_loop(((i + 1) * _T) // _C, _NC, col_body, 0)
        return carry

    jax.lax.fori_loop(0, _NB, block_body, 0)


def kernel(loc, conf, iou):
    b = loc.shape[0]
    loc_t = jnp.transpose(loc, (0, 2, 1))
    loc_t = jnp.pad(loc_t, ((0, 0), (0, 0), (0, _NP_PAD - _NP)))
    conf1 = jnp.pad(conf[:, :, 1:2], ((0, 0), (0, _NP_PAD - _NP), (0, 0)))
    conf1 = jnp.transpose(conf1, (0, 2, 1))
    iou0 = jnp.pad(iou[:, :, 0:1], ((0, 0), (0, _NP_PAD - _NP), (0, 0)))
    iou0 = jnp.transpose(iou0, (0, 2, 1))
    priors = jnp.asarray(_PRIORS_T)

    boxes_t, scores = pl.pallas_call(
        _decode_kernel,
        grid=(b,),
        in_specs=[
            pl.BlockSpec((None, 14, _NP_PAD), lambda i: (i, 0, 0)),
            pl.BlockSpec((4, _NP_PAD), lambda i: (0, 0)),
            pl.BlockSpec((None, 1, _NP_PAD), lambda i: (i, 0, 0)),
            pl.BlockSpec((None, 1, _NP_PAD), lambda i: (i, 0, 0)),
        ],
        out_specs=[
            pl.BlockSpec((None, 14, _NP_PAD), lambda i: (i, 0, 0)),
            pl.BlockSpec((None, 1, _NP_PAD), lambda i: (i, 0, 0)),
        ],
        out_shape=[
            jax.ShapeDtypeStruct((b, 14, _NP_PAD), jnp.float32),
            jax.ShapeDtypeStruct((b, 1, _NP_PAD), jnp.float32),
        ],
    )(loc_t, priors, conf1, iou0)

    scores = scores[:, 0, :_NP]
    top_scores, order = jax.lax.top_k(scores, _TOP_K)
    boxes_full = jnp.transpose(boxes_t, (0, 2, 1))[:, :_NP, :]
    boxes_k = jnp.take_along_axis(boxes_full, order[:, :, None], axis=1)

    rows = jnp.pad(boxes_k[:, :, :4], ((0, 0), (0, _N_PAD - _TOP_K), (0, 0)))
    cols = jnp.transpose(rows, (0, 2, 1))
    valid = (top_scores > 0.0).astype(jnp.float32)
    valid = jnp.pad(valid, ((0, 0), (0, _N_PAD - _TOP_K)))[:, None, :]

    keep = pl.pallas_call(
        _nms_kernel,
        grid=(b,),
        in_specs=[
            pl.BlockSpec((None, _N_PAD, 4), lambda i: (i, 0, 0)),
            pl.BlockSpec((None, 4, _N_PAD), lambda i: (i, 0, 0)),
            pl.BlockSpec((None, 1, _N_PAD), lambda i: (i, 0, 0)),
        ],
        out_specs=pl.BlockSpec((None, 1, _N_PAD), lambda i: (i, 0, 0)),
        out_shape=jax.ShapeDtypeStruct((b, 1, _N_PAD), jnp.float32),
    )(rows, cols, valid)

    kept = keep[:, 0, :_TOP_K] > 0.5
    final_scores = jnp.where(kept, top_scores, -1.0)
    fs, fidx = jax.lax.top_k(final_scores, _KEEP_TOP_K)
    dets = jnp.take_along_axis(boxes_k, fidx[:, :, None], axis=1)
    sc = jnp.take_along_axis(top_scores, fidx, axis=1)
    out = jnp.concatenate([dets, sc[:, :, None]], axis=-1)
    return out * (fs > 0.0).astype(out.dtype)[:, :, None]


# speedup vs baseline: 256.0765x; 1.2011x over previous
import math

import numpy as np
import jax
import jax.numpy as jnp
from jax.experimental import pallas as pl
from jax.experimental.pallas import tpu as pltpu

_H = 512
_W = 512
_MIN_SIZES = [[10, 16, 24], [32, 48], [64, 96], [128, 192, 256]]
_STEPS = [8, 16, 32, 64]
_VAR0 = 0.1
_VAR1 = 0.2
_TOP_K = 5000
_CONF_THR = 0.3
_NMS_THR = 0.3
_KEEP_TOP_K = 750

_NP = sum(math.ceil(_H / s) * math.ceil(_W / s) * len(m)
          for s, m in zip(_STEPS, _MIN_SIZES))
_NP_PAD = ((_NP + 127) // 128) * 128
_T = 256
_N_PAD = ((_TOP_K + _T - 1) // _T) * _T
_NB = _N_PAD // _T
_C = 512
_NC = _N_PAD // _C


def _build_priors_np():
    anchors = []
    for k, step in enumerate(_STEPS):
        fh = math.ceil(_H / step)
        fw = math.ceil(_W / step)
        for i in range(fh):
            for j in range(fw):
                for ms in _MIN_SIZES[k]:
                    anchors.append([(j + 0.5) * step / _W, (i + 0.5) * step / _H,
                                    ms / _W, ms / _H])
    return np.array(anchors, dtype=np.float32)


_PRIORS_T = np.zeros((4, _NP_PAD), dtype=np.float32)
_PRIORS_T[:, :_NP] = _build_priors_np().T


def _decode_kernel(loc_ref, pri_ref, conf_ref, iou_ref, box_ref, sc_ref):
    loc = loc_ref[...]
    pxy = pri_ref[0:2, :]
    pwh = pri_ref[2:4, :]
    cxy = pxy + loc[0:2, :] * _VAR0 * pwh
    wh = pwh * jnp.exp(loc[2:4, :] * _VAR1)
    x1y1 = cxy - wh * 0.5
    x2y2 = x1y1 + wh
    lmks = [pxy + loc[s:s + 2, :] * _VAR0 * pwh for s in (4, 6, 8, 10, 12)]
    box_ref[...] = jnp.concatenate([x1y1, x2y2] + lmks, axis=0) * float(_W)
    cls = conf_ref[...]
    iouv = jnp.clip(iou_ref[...], 0.0, 1.0)
    s = jnp.sqrt(jnp.maximum(cls * iouv, 1e-12))
    sc_ref[...] = jnp.where(s > _CONF_THR, s, -1.0)


def _nms_kernel(rows_ref, cols_ref, valid_ref, keep_ref):
    x1 = cols_ref[0:1, :]
    y1 = cols_ref[1:2, :]
    x2 = cols_ref[2:3, :]
    y2 = cols_ref[3:4, :]
    iota_c = jax.lax.broadcasted_iota(jnp.int32, (1, _C), 1)
    rt = jax.lax.broadcasted_iota(jnp.int32, (_T, _T), 0)
    ct = jax.lax.broadcasted_iota(jnp.int32, (_T, _T), 1)
    ltmask = (rt < ct).astype(jnp.float32)
    keep_ref[...] = valid_ref[...]

    def block_body(i, carry):
        bb = rows_ref[pl.dslice(i * _T, _T), :]
        x1b = bb[:, 0:1]
        y1b = bb[:, 1:2]
        x2b = bb[:, 2:3]
        y2b = bb[:, 3:4]
        ab = (x2b - x1b) * (y2b - y1b)

        x1r = cols_ref[0:1, pl.dslice(i * _T, _T)]
        y1r = cols_ref[1:2, pl.dslice(i * _T, _T)]
        x2r = cols_ref[2:3, pl.dslice(i * _T, _T)]
        y2r = cols_ref[3:4, pl.dslice(i * _T, _T)]
        ar = (x2r - x1r) * (y2r - y1r)
        interb = (jnp.maximum(jnp.minimum(x2b, x2r) - jnp.maximum(x1b, x1r), 0.0)
                  * jnp.maximum(jnp.minimum(y2b, y2r) - jnp.maximum(y1b, y1r), 0.0))
        ioub = interb / (ab + ar - interb + 1e-12)
        mlt = (ioub > _NMS_THR).astype(jnp.float32) * ltmask
        kb0 = keep_ref[0:1, pl.dslice(i * _T, _T)]

        def fcond(carry):
            return carry[1]

        def fbody(carry):
            kb, _ = carry
            sup = jnp.dot(kb, mlt, preferred_element_type=jnp.float32)
            knew = kb0 * jnp.where(sup > 0.0, 0.0, 1.0)
            return knew, jnp.any(knew != kb)

        kb, _ = jax.lax.while_loop(fcond, fbody, (kb0, True))
        keep_ref[0:1, pl.dslice(i * _T, _T)] = kb

        def col_body(j, c2):
            x1c = cols_ref[0:1, pl.dslice(j * _C, _C)]
            y1c = cols_ref[1:2, pl.dslice(j * _C, _C)]
            x2c = cols_ref[2:3, pl.dslice(j * _C, _C)]
            y2c = cols_ref[3:4, pl.dslice(j * _C, _C)]
            ac = (x2c - x1c) * (y2c - y1c)
            ic = (jnp.maximum(jnp.minimum(x2b, x2c) - jnp.maximum(x1b, x1c), 0.0)
                  * jnp.maximum(jnp.minimum(y2b, y2c) - jnp.maximum(y1b, y1c), 0.0))
            iouc = ic / (ab + ac - ic + 1e-12)
            mc = (iouc > _NMS_THR).astype(jnp.float32)
            cross = jnp.dot(kb, mc, preferred_element_type=jnp.float32)
            sup = (cross > 0.0) & (iota_c + j * _C >= (i + 1) * _T)
            keep_ref[0:1, pl.dslice(j * _C, _C)] = (
                keep_ref[0:1, pl.dslice(j * _C, _C)]
                * (1.0 - sup.astype(jnp.float32)))
            return c2

        jax.lax.fori_loop(((i + 1) * _T) // _C, _NC, col_body, 0)
        return carry

    jax.lax.fori_loop(0, _NB, block_body, 0)


def kernel(loc, conf, iou):
    b = loc.shape[0]
    loc_t = jnp.transpose(loc, (0, 2, 1))
    loc_t = jnp.pad(loc_t, ((0, 0), (0, 0), (0, _NP_PAD - _NP)))
    conf1 = jnp.pad(conf[:, :, 1:2], ((0, 0), (0, _NP_PAD - _NP), (0, 0)))
    conf1 = jnp.transpose(conf1, (0, 2, 1))
    iou0 = jnp.pad(iou[:, :, 0:1], ((0, 0), (0, _NP_PAD - _NP), (0, 0)))
    iou0 = jnp.transpose(iou0, (0, 2, 1))
    priors = jnp.asarray(_PRIORS_T)

    boxes_t, scores = pl.pallas_call(
        _decode_kernel,
        grid=(b,),
        in_specs=[
            pl.BlockSpec((None, 14, _NP_PAD), lambda i: (i, 0, 0)),
            pl.BlockSpec((4, _NP_PAD), lambda i: (0, 0)),
            pl.BlockSpec((None, 1, _NP_PAD), lambda i: (i, 0, 0)),
            pl.BlockSpec((None, 1, _NP_PAD), lambda i: (i, 0, 0)),
        ],
        out_specs=[
            pl.BlockSpec((None, 14, _NP_PAD), lambda i: (i, 0, 0)),
            pl.BlockSpec((None, 1, _NP_PAD), lambda i: (i, 0, 0)),
        ],
        out_shape=[
            jax.ShapeDtypeStruct((b, 14, _NP_PAD), jnp.float32),
            jax.ShapeDtypeStruct((b, 1, _NP_PAD), jnp.float32),
        ],
    )(loc_t, priors, conf1, iou0)

    scores = scores[:, 0, :_NP]
    top_scores, order = jax.lax.top_k(scores, _TOP_K)
    boxes_full = jnp.transpose(boxes_t, (0, 2, 1))[:, :_NP, :]
    boxes_k = jnp.take_along_axis(boxes_full, order[:, :, None], axis=1)

    rows = jnp.pad(boxes_k[:, :, :4], ((0, 0), (0, _N_PAD - _TOP_K), (0, 0)))
    cols = jnp.transpose(rows, (0, 2, 1))
    valid = (top_scores > 0.0).astype(jnp.float32)
    valid = jnp.pad(valid, ((0, 0), (0, _N_PAD - _TOP_K)))[:, None, :]

    keep = pl.pallas_call(
        _nms_kernel,
        grid=(b,),
        in_specs=[
            pl.BlockSpec((None, _N_PAD, 4), lambda i: (i, 0, 0)),
            pl.BlockSpec((None, 4, _N_PAD), lambda i: (i, 0, 0)),
            pl.BlockSpec((None, 1, _N_PAD), lambda i: (i, 0, 0)),
        ],
        out_specs=pl.BlockSpec((None, 1, _N_PAD), lambda i: (i, 0, 0)),
        out_shape=jax.ShapeDtypeStruct((b, 1, _N_PAD), jnp.float32),
    )(rows, cols, valid)

    kept = keep[:, 0, :_TOP_K] > 0.5
    final_scores = jnp.where(kept, top_scores, -1.0)
    fs, fidx = jax.lax.top_k(final_scores, _KEEP_TOP_K)
    dets = jnp.take_along_axis(boxes_k, fidx[:, :, None], axis=1)
    sc = jnp.take_along_axis(top_scores, fidx, axis=1)
    out = jnp.concatenate([dets, sc[:, :, None]], axis=-1)
    return out * (fs > 0.0).astype(out.dtype)[:, :, None]
